# baseline (speedup 1.0000x reference)
import math

import jax
import jax.numpy as jnp
from jax.experimental import pallas as pl
from jax.experimental.pallas import tpu as pltpu

C = 384
NH = 8
HD = C // NH
WIN = 64
TB = 1792
N = 50176
GRID = N // TB
TOPK = 64
SCALE = 1.0 / math.sqrt(HD)
NEG = -3.0e38


def _ln(t, w, b):
    u = jnp.mean(t, axis=1, keepdims=True)
    s = jnp.mean((t - u) ** 2, axis=1, keepdims=True)
    return (t - u) * jax.lax.rsqrt(s + 1e-6) * w + b


def _local_kernel(x_ref, wqkv_ref, bqkv_ref, n1w_ref, n1b_ref, wlo_ref,
                  blo_ref, n2w_ref, n2b_ref, wqg_ref, bqg_ref,
                  x1_ref, score_ref, smap_ref):
    x = jnp.transpose(x_ref[...])
    xn = _ln(x, n1w_ref[...], n1b_ref[...])
    qkv = jnp.dot(xn.astype(jnp.bfloat16), wqkv_ref[...],
                  preferred_element_type=jnp.float32) + bqkv_ref[...]
    qkv = qkv.astype(jnp.bfloat16).reshape(8, TB // WIN, 8, 3 * C)
    qkv = jnp.transpose(qkv, (1, 0, 2, 3)).reshape(TB // WIN, WIN, 3 * C)
    acc = jnp.zeros((TB, C), jnp.float32)
    for h in range(NH):
        q = qkv[:, :, h * HD:(h + 1) * HD]
        k = qkv[:, :, C + h * HD:C + (h + 1) * HD]
        v = qkv[:, :, 2 * C + h * HD:2 * C + (h + 1) * HD]
        logits = jax.lax.dot_general(
            q, k, (((2,), (2,)), ((0,), (0,))),
            preferred_element_type=jnp.float32) * SCALE
        a = jax.nn.softmax(logits, axis=-1).astype(jnp.bfloat16)
        o = jax.lax.dot_general(
            a, v, (((2,), (1,)), ((0,), (0,))),
            preferred_element_type=jnp.float32)
        acc = acc + jnp.dot(o.reshape(TB, HD).astype(jnp.bfloat16),
                            wlo_ref[h * HD:(h + 1) * HD, :],
                            preferred_element_type=jnp.float32)
    acc = acc.reshape(TB // WIN, 8, 8, C)
    acc = jnp.transpose(acc, (1, 0, 2, 3)).reshape(TB, C)
    x1 = x + acc + blo_ref[...]
    x1_ref[...] = x1
    xn2 = _ln(x1, n2w_ref[...], n2b_ref[...])
    score = jnp.dot(xn2, wqg_ref[...],
                    preferred_element_type=jnp.float32) + bqg_ref[...]
    score_ref[...] = score
    smap_ref[...] = jnp.sum(score * score, axis=1).reshape(1, TB // 128, 128)


def _topk_kernel(smap_ref, idx_ref):
    s = smap_ref[...]
    rows, lanes = s.shape
    lin = (jax.lax.broadcasted_iota(jnp.int32, (rows, lanes), 0) * lanes
           + jax.lax.broadcasted_iota(jnp.int32, (rows, lanes), 1))

    def body(i, s):
        m = jnp.max(s)
        cand = jnp.where(s >= m, lin, jnp.int32(2 ** 30))
        j = jnp.min(cand)
        idx_ref[0, i] = j
        return jnp.where(lin == j, NEG, s)

    jax.lax.fori_loop(0, TOPK, body, s)


def _gather_kernel(idx_ref, x1_ref, out_ref):
    del idx_ref
    out_ref[...] = x1_ref[...]


def _global_kernel(score_ref, x1_ref, xg_ref, n2w_ref, n2b_ref, wagq_ref,
                   bagq_ref, wkv_ref, bkv_ref, wgo_ref, bgo_ref, n3w_ref,
                   n3b_ref, wf1_ref, bf1_ref, wf2_ref, bf2_ref, out_ref):
    score = score_ref[...]
    qh = jnp.dot(score.astype(jnp.bfloat16), wagq_ref[...],
                 preferred_element_type=jnp.float32) + bagq_ref[...]
    qh = qh.astype(jnp.bfloat16)
    xgn = _ln(xg_ref[...], n2w_ref[...], n2b_ref[...])
    kv = jnp.dot(xgn.astype(jnp.bfloat16), wkv_ref[...],
                 preferred_element_type=jnp.float32) + bkv_ref[...]
    kv = kv.astype(jnp.bfloat16)
    acc = jnp.zeros((TB, C), jnp.float32)
    for h in range(NH):
        q_h = qh[:, h * HD:(h + 1) * HD]
        k_h = kv[:, h * HD:(h + 1) * HD]
        v_h = kv[:, C + h * HD:C + (h + 1) * HD]
        logits = jax.lax.dot_general(
            q_h, k_h, (((1,), (1,)), ((), ())),
            preferred_element_type=jnp.float32) * SCALE
        a = jax.nn.softmax(logits, axis=-1).astype(jnp.bfloat16)
        o = jnp.dot(a, v_h, preferred_element_type=jnp.float32)
        acc = acc + jnp.dot(o.astype(jnp.bfloat16),
                            wgo_ref[h * HD:(h + 1) * HD, :],
                            preferred_element_type=jnp.float32)
    x2 = x1_ref[...] + acc + bgo_ref[...]
    xn3 = _ln(x2, n3w_ref[...], n3b_ref[...])
    f = jnp.dot(xn3.astype(jnp.bfloat16), wf1_ref[...],
                preferred_element_type=jnp.float32) + bf1_ref[...]
    f = 0.5 * f * (1.0 + jax.lax.erf(f * (1.0 / math.sqrt(2.0))))
    y = x2 + jnp.dot(f.astype(jnp.bfloat16), wf2_ref[...],
                     preferred_element_type=jnp.float32) + bf2_ref[...]
    out_ref[...] = jnp.transpose(y)


def kernel(x, n1_w, n1_b, qkv_w, qkv_b, al_in_w, al_in_b, al_out_w, al_out_b,
           pl_w, pl_b, n2_w, n2_b, qg_w, qg_b, kvg_w, kvg_b, ag_in_w, ag_in_b,
           ag_out_w, ag_out_b, pg_w, pg_b, n3_w, n3_b, ffn1_w, ffn1_b,
           ffn2_w, ffn2_b):
    B, Cc, H, W = x.shape
    nh_w = H // 8

    wq = al_in_w[:C] @ qkv_w[:C]
    wk = al_in_w[C:2 * C] @ qkv_w[C:2 * C]
    wv = al_in_w[2 * C:] @ qkv_w[2 * C:]
    wqkv = jnp.concatenate([wq, wk, wv], axis=0).T
    bqkv = jnp.concatenate([
        al_in_w[:C] @ qkv_b[:C] + al_in_b[:C],
        al_in_w[C:2 * C] @ qkv_b[C:2 * C] + al_in_b[C:2 * C],
        al_in_w[2 * C:] @ qkv_b[2 * C:] + al_in_b[2 * C:]])[None]
    wlo = (pl_w @ al_out_w).T
    blo = (pl_w @ al_out_b + pl_b)[None]
    wqg = qg_w.T
    bqg = qg_b[None]
    wagq = ag_in_w[:C].T
    bagq = ag_in_b[:C][None]
    wkv = jnp.concatenate([(ag_in_w[C:2 * C] @ kvg_w[:C]).T,
                           (ag_in_w[2 * C:] @ kvg_w[C:]).T], axis=1)
    bkv = jnp.concatenate([
        ag_in_w[C:2 * C] @ kvg_b[:C] + ag_in_b[C:2 * C],
        ag_in_w[2 * C:] @ kvg_b[C:] + ag_in_b[2 * C:]])[None]
    wgo = (pg_w @ ag_out_w).T
    bgo = (pg_w @ ag_out_b + pg_b)[None]
    wf1 = ffn1_w.T
    bf1 = ffn1_b[None]
    wf2 = ffn2_w.T
    bf2 = ffn2_b[None]
    bf = jnp.bfloat16
    wqkv = wqkv.astype(bf)
    wlo = wlo.astype(bf)
    wagq = wagq.astype(bf)
    wkv = wkv.astype(bf)
    wgo = wgo.astype(bf)
    wf1 = wf1.astype(bf)
    wf2 = wf2.astype(bf)
    n1w, n1b = n1_w[None], n1_b[None]
    n2w, n2b = n2_w[None], n2_b[None]
    n3w, n3b = n3_w[None], n3_b[None]

    xp = x[0].reshape(C, N)

    def full(shape):
        return pl.BlockSpec(shape, lambda i: (0,) * len(shape))

    tok = pl.BlockSpec((TB, C), lambda i: (i, 0))
    cmaj = pl.BlockSpec((C, TB), lambda i: (0, i))

    x1, score, smap = pl.pallas_call(
        _local_kernel,
        grid=(GRID,),
        in_specs=[cmaj, full((C, 3 * C)), full((1, 3 * C)), full((1, C)),
                  full((1, C)), full((C, C)), full((1, C)), full((1, C)),
                  full((1, C)), full((C, C)), full((1, C))],
        out_specs=[tok, pl.BlockSpec((TB, C), lambda i: (i, 0)),
                   pl.BlockSpec((1, TB // 128, 128), lambda i: (i, 0, 0))],
        out_shape=[jax.ShapeDtypeStruct((N, C), jnp.float32),
                   jax.ShapeDtypeStruct((N, C), jnp.float32),
                   jax.ShapeDtypeStruct((GRID, TB // 128, 128), jnp.float32)],
    )(xp, wqkv, bqkv, n1w, n1b, wlo, blo, n2w, n2b, wqg, bqg)
    smap = smap.reshape(N // 128, 128)

    idx = pl.pallas_call(
        _topk_kernel,
        in_specs=[pl.BlockSpec(memory_space=pltpu.VMEM)],
        out_specs=pl.BlockSpec(memory_space=pltpu.SMEM),
        out_shape=jax.ShapeDtypeStruct((1, TOPK), jnp.int32),
    )(smap)

    xg = pl.pallas_call(
        _gather_kernel,
        grid_spec=pltpu.PrefetchScalarGridSpec(
            num_scalar_prefetch=1,
            grid=(TOPK,),
            in_specs=[pl.BlockSpec(
                (1, 1, C), lambda i, idx_ref: (idx_ref[0, i], 0, 0))],
            out_specs=pl.BlockSpec((1, 1, C), lambda i, idx_ref: (i, 0, 0)),
        ),
        out_shape=jax.ShapeDtypeStruct((TOPK, 1, C), jnp.float32),
    )(idx, x1.reshape(N, 1, C))
    xg = xg.reshape(TOPK, C)

    y = pl.pallas_call(
        _global_kernel,
        grid=(GRID,),
        in_specs=[tok, pl.BlockSpec((TB, C), lambda i: (i, 0)),
                  full((TOPK, C)), full((1, C)), full((1, C)),
                  full((C, C)), full((1, C)), full((C, 2 * C)),
                  full((1, 2 * C)), full((C, C)), full((1, C)),
                  full((1, C)), full((1, C)), full((C, 2 * C)),
                  full((1, 2 * C)), full((2 * C, C)), full((1, C))],
        out_specs=cmaj,
        out_shape=jax.ShapeDtypeStruct((C, N), jnp.float32),
    )(score, x1, xg, n2w, n2b, wagq, bagq, wkv, bkv, wgo, bgo,
      n3w, n3b, wf1, bf1, wf2, bf2)

    return y.reshape(1, C, H, W)


# speedup vs baseline: 1.1701x; 1.1701x over previous
import math

import jax
import jax.numpy as jnp
from jax.experimental import pallas as pl
from jax.experimental.pallas import tpu as pltpu

C = 384
NH = 8
HD = C // NH
WIN = 64
TB = 3584
N = 50176
GRID = N // TB
TOPK = 64
SCALE = 1.0 / math.sqrt(HD)
NEG = -3.0e38


def _ln(t, w, b):
    u = jnp.mean(t, axis=1, keepdims=True)
    s = jnp.mean((t - u) ** 2, axis=1, keepdims=True)
    return (t - u) * jax.lax.rsqrt(s + 1e-6) * w + b


def _local_kernel(x_ref, wqkv_ref, bqkv_ref, n1w_ref, n1b_ref, wlo_ref,
                  blo_ref, n2w_ref, n2b_ref, wqg_ref, bqg_ref,
                  x1_ref, score_ref, smap_ref):
    x = x_ref[...]
    xn = _ln(x, n1w_ref[...], n1b_ref[...])
    qkv = jnp.dot(xn.astype(jnp.bfloat16), wqkv_ref[...],
                  preferred_element_type=jnp.float32) + bqkv_ref[...]
    qkv = qkv.astype(jnp.bfloat16)
    acc = jnp.zeros((TB, C), jnp.float32)
    for h in range(NH):
        q = qkv[:, h * HD:(h + 1) * HD].reshape(TB // WIN, WIN, HD)
        k = qkv[:, C + h * HD:C + (h + 1) * HD].reshape(TB // WIN, WIN, HD)
        v = qkv[:, 2 * C + h * HD:2 * C + (h + 1) * HD].reshape(
            TB // WIN, WIN, HD)
        logits = jax.lax.dot_general(
            q, k, (((2,), (2,)), ((0,), (0,))),
            preferred_element_type=jnp.float32) * SCALE
        a = jax.nn.softmax(logits, axis=-1).astype(jnp.bfloat16)
        o = jax.lax.dot_general(
            a, v, (((2,), (1,)), ((0,), (0,))),
            preferred_element_type=jnp.float32)
        acc = acc + jnp.dot(o.reshape(TB, HD).astype(jnp.bfloat16),
                            wlo_ref[h * HD:(h + 1) * HD, :],
                            preferred_element_type=jnp.float32)
    x1 = x + acc + blo_ref[...]
    x1_ref[...] = x1
    xn2 = _ln(x1, n2w_ref[...], n2b_ref[...])
    score = jnp.dot(xn2, wqg_ref[...],
                    preferred_element_type=jnp.float32) + bqg_ref[...]
    score_ref[...] = score
    smap_ref[...] = jnp.sum(score * score, axis=1).reshape(1, TB // 128, 128)


def _topk_kernel(smap_ref, idx_ref):
    s = smap_ref[...]
    rows, lanes = s.shape
    lin = (jax.lax.broadcasted_iota(jnp.int32, (rows, lanes), 0) * lanes
           + jax.lax.broadcasted_iota(jnp.int32, (rows, lanes), 1))

    def body(i, s):
        m = jnp.max(s)
        cand = jnp.where(s >= m, lin, jnp.int32(2 ** 30))
        j = jnp.min(cand)
        idx_ref[0, i] = j
        return jnp.where(lin == j, NEG, s)

    jax.lax.fori_loop(0, TOPK, body, s)


def _gather_kernel(idx_ref, x1_ref, out_ref):
    del idx_ref
    out_ref[...] = x1_ref[...]


def _global_kernel(score_ref, x1_ref, xg_ref, n2w_ref, n2b_ref, wagq_ref,
                   bagq_ref, wkv_ref, bkv_ref, wgo_ref, bgo_ref, n3w_ref,
                   n3b_ref, wf1_ref, bf1_ref, wf2_ref, bf2_ref, out_ref):
    score = score_ref[...]
    qh = jnp.dot(score.astype(jnp.bfloat16), wagq_ref[...],
                 preferred_element_type=jnp.float32) + bagq_ref[...]
    qh = qh.astype(jnp.bfloat16)
    xgn = _ln(xg_ref[...], n2w_ref[...], n2b_ref[...])
    kv = jnp.dot(xgn.astype(jnp.bfloat16), wkv_ref[...],
                 preferred_element_type=jnp.float32) + bkv_ref[...]
    kv = kv.astype(jnp.bfloat16)
    acc = jnp.zeros((TB, C), jnp.float32)
    for h in range(NH):
        q_h = qh[:, h * HD:(h + 1) * HD]
        k_h = kv[:, h * HD:(h + 1) * HD]
        v_h = kv[:, C + h * HD:C + (h + 1) * HD]
        logits = jax.lax.dot_general(
            q_h, k_h, (((1,), (1,)), ((), ())),
            preferred_element_type=jnp.float32) * SCALE
        a = jax.nn.softmax(logits, axis=-1).astype(jnp.bfloat16)
        o = jnp.dot(a, v_h, preferred_element_type=jnp.float32)
        acc = acc + jnp.dot(o.astype(jnp.bfloat16),
                            wgo_ref[h * HD:(h + 1) * HD, :],
                            preferred_element_type=jnp.float32)
    x2 = x1_ref[...] + acc + bgo_ref[...]
    xn3 = _ln(x2, n3w_ref[...], n3b_ref[...])
    f = jnp.dot(xn3.astype(jnp.bfloat16), wf1_ref[...],
                preferred_element_type=jnp.float32) + bf1_ref[...]
    f = 0.5 * f * (1.0 + jax.lax.erf(f * (1.0 / math.sqrt(2.0))))
    out_ref[...] = x2 + jnp.dot(f.astype(jnp.bfloat16), wf2_ref[...],
                                preferred_element_type=jnp.float32) + bf2_ref[...]


def kernel(x, n1_w, n1_b, qkv_w, qkv_b, al_in_w, al_in_b, al_out_w, al_out_b,
           pl_w, pl_b, n2_w, n2_b, qg_w, qg_b, kvg_w, kvg_b, ag_in_w, ag_in_b,
           ag_out_w, ag_out_b, pg_w, pg_b, n3_w, n3_b, ffn1_w, ffn1_b,
           ffn2_w, ffn2_b):
    B, Cc, H, W = x.shape
    nh_w = H // 8

    wq = al_in_w[:C] @ qkv_w[:C]
    wk = al_in_w[C:2 * C] @ qkv_w[C:2 * C]
    wv = al_in_w[2 * C:] @ qkv_w[2 * C:]
    wqkv = jnp.concatenate([wq, wk, wv], axis=0).T
    bqkv = jnp.concatenate([
        al_in_w[:C] @ qkv_b[:C] + al_in_b[:C],
        al_in_w[C:2 * C] @ qkv_b[C:2 * C] + al_in_b[C:2 * C],
        al_in_w[2 * C:] @ qkv_b[2 * C:] + al_in_b[2 * C:]])[None]
    wlo = (pl_w @ al_out_w).T
    blo = (pl_w @ al_out_b + pl_b)[None]
    wqg = qg_w.T
    bqg = qg_b[None]
    wagq = ag_in_w[:C].T
    bagq = ag_in_b[:C][None]
    wkv = jnp.concatenate([(ag_in_w[C:2 * C] @ kvg_w[:C]).T,
                           (ag_in_w[2 * C:] @ kvg_w[C:]).T], axis=1)
    bkv = jnp.concatenate([
        ag_in_w[C:2 * C] @ kvg_b[:C] + ag_in_b[C:2 * C],
        ag_in_w[2 * C:] @ kvg_b[C:] + ag_in_b[2 * C:]])[None]
    wgo = (pg_w @ ag_out_w).T
    bgo = (pg_w @ ag_out_b + pg_b)[None]
    wf1 = ffn1_w.T
    bf1 = ffn1_b[None]
    wf2 = ffn2_w.T
    bf2 = ffn2_b[None]
    bf = jnp.bfloat16
    wqkv = wqkv.astype(bf)
    wlo = wlo.astype(bf)
    wagq = wagq.astype(bf)
    wkv = wkv.astype(bf)
    wgo = wgo.astype(bf)
    wf1 = wf1.astype(bf)
    wf2 = wf2.astype(bf)
    n1w, n1b = n1_w[None], n1_b[None]
    n2w, n2b = n2_w[None], n2_b[None]
    n3w, n3b = n3_w[None], n3_b[None]

    xp = x[0].reshape(C, nh_w, 8, nh_w, 8).transpose(1, 3, 2, 4, 0)
    xp = xp.reshape(N, C)

    def full(shape):
        return pl.BlockSpec(shape, lambda i: (0,) * len(shape))

    tok = pl.BlockSpec((TB, C), lambda i: (i, 0))

    x1, score, smap = pl.pallas_call(
        _local_kernel,
        grid=(GRID,),
        in_specs=[tok, full((C, 3 * C)), full((1, 3 * C)), full((1, C)),
                  full((1, C)), full((C, C)), full((1, C)), full((1, C)),
                  full((1, C)), full((C, C)), full((1, C))],
        out_specs=[tok, pl.BlockSpec((TB, C), lambda i: (i, 0)),
                   pl.BlockSpec((1, TB // 128, 128), lambda i: (i, 0, 0))],
        out_shape=[jax.ShapeDtypeStruct((N, C), jnp.float32),
                   jax.ShapeDtypeStruct((N, C), jnp.float32),
                   jax.ShapeDtypeStruct((GRID, TB // 128, 128), jnp.float32)],
    )(xp, wqkv, bqkv, n1w, n1b, wlo, blo, n2w, n2b, wqg, bqg)
    smap = smap.reshape(N // 128, 128)

    idx = pl.pallas_call(
        _topk_kernel,
        in_specs=[pl.BlockSpec(memory_space=pltpu.VMEM)],
        out_specs=pl.BlockSpec(memory_space=pltpu.SMEM),
        out_shape=jax.ShapeDtypeStruct((1, TOPK), jnp.int32),
    )(smap)

    xg = pl.pallas_call(
        _gather_kernel,
        grid_spec=pltpu.PrefetchScalarGridSpec(
            num_scalar_prefetch=1,
            grid=(TOPK,),
            in_specs=[pl.BlockSpec(
                (1, 1, C), lambda i, idx_ref: (idx_ref[0, i], 0, 0))],
            out_specs=pl.BlockSpec((1, 1, C), lambda i, idx_ref: (i, 0, 0)),
        ),
        out_shape=jax.ShapeDtypeStruct((TOPK, 1, C), jnp.float32),
    )(idx, x1.reshape(N, 1, C))
    xg = xg.reshape(TOPK, C)

    y = pl.pallas_call(
        _global_kernel,
        grid=(GRID,),
        in_specs=[tok, pl.BlockSpec((TB, C), lambda i: (i, 0)),
                  full((TOPK, C)), full((1, C)), full((1, C)),
                  full((C, C)), full((1, C)), full((C, 2 * C)),
                  full((1, 2 * C)), full((C, C)), full((1, C)),
                  full((1, C)), full((1, C)), full((C, 2 * C)),
                  full((1, 2 * C)), full((2 * C, C)), full((1, C))],
        out_specs=pl.BlockSpec((TB, C), lambda i: (i, 0)),
        out_shape=jax.ShapeDtypeStruct((N, C), jnp.float32),
    )(score, x1, xg, n2w, n2b, wagq, bagq, wkv, bkv, wgo, bgo,
      n3w, n3b, wf1, bf1, wf2, bf2)

    out = y.reshape(nh_w, nh_w, 8, 8, C).transpose(4, 0, 2, 1, 3)
    return out.reshape(1, C, H, W)


# speedup vs baseline: 1.2867x; 1.0996x over previous
import functools
import math

import jax
import jax.numpy as jnp
from jax import lax
from jax.experimental import pallas as pl
from jax.experimental.pallas import tpu as pltpu
from jax.experimental.pallas import tpu_sc as plsc

C = 384
NH = 8
HD = C // NH
WIN = 64
TB = 3584
N = 50176
GRID = N // TB
TOPK = 64
SCALE = 1.0 / math.sqrt(HD)
NEG = -3.0e38


def _ln(t, w, b):
    u = jnp.mean(t, axis=1, keepdims=True)
    s = jnp.mean((t - u) ** 2, axis=1, keepdims=True)
    return (t - u) * jax.lax.rsqrt(s + 1e-6) * w + b


def _local_kernel(x_ref, wqkv_ref, bqkv_ref, n1w_ref, n1b_ref, wlo_ref,
                  blo_ref, n2w_ref, n2b_ref, wqg_ref, bqg_ref,
                  x1_ref, score_ref, smap_ref):
    x = x_ref[...]
    xn = _ln(x, n1w_ref[...], n1b_ref[...])
    qkv = jnp.dot(xn.astype(jnp.bfloat16), wqkv_ref[...],
                  preferred_element_type=jnp.float32) + bqkv_ref[...]
    qkv = qkv.astype(jnp.bfloat16)
    acc = jnp.zeros((TB, C), jnp.float32)
    for h in range(NH):
        q = qkv[:, h * HD:(h + 1) * HD].reshape(TB // WIN, WIN, HD)
        k = qkv[:, C + h * HD:C + (h + 1) * HD].reshape(TB // WIN, WIN, HD)
        v = qkv[:, 2 * C + h * HD:2 * C + (h + 1) * HD].reshape(
            TB // WIN, WIN, HD)
        logits = jax.lax.dot_general(
            q, k, (((2,), (2,)), ((0,), (0,))),
            preferred_element_type=jnp.float32) * SCALE
        a = jax.nn.softmax(logits, axis=-1).astype(jnp.bfloat16)
        o = jax.lax.dot_general(
            a, v, (((2,), (1,)), ((0,), (0,))),
            preferred_element_type=jnp.float32)
        acc = acc + jnp.dot(o.reshape(TB, HD).astype(jnp.bfloat16),
                            wlo_ref[h * HD:(h + 1) * HD, :],
                            preferred_element_type=jnp.float32)
    x1 = x + acc + blo_ref[...]
    x1_ref[...] = x1
    xn2 = _ln(x1, n2w_ref[...], n2b_ref[...])
    score = jnp.dot(xn2, wqg_ref[...],
                    preferred_element_type=jnp.float32) + bqg_ref[...]
    score_ref[...] = score
    smap_ref[...] = jnp.sum(score * score, axis=1).reshape(1, TB // 128, 128)


def _topk_kernel(smap_ref, idx_ref):
    s = smap_ref[...]
    rows, lanes = s.shape
    lin = (jax.lax.broadcasted_iota(jnp.int32, (rows, lanes), 0) * lanes
           + jax.lax.broadcasted_iota(jnp.int32, (rows, lanes), 1))

    def body(i, s):
        m = jnp.max(s)
        cand = jnp.where(s >= m, lin, jnp.int32(2 ** 30))
        j = jnp.min(cand)
        idx_ref[0, i] = j
        return jnp.where(lin == j, NEG, s)

    jax.lax.fori_loop(0, TOPK, body, s)


_SC_WORKERS = 8
_ROWS_PER_W = TOPK // _SC_WORKERS


@functools.partial(
    pl.kernel,
    mesh=plsc.VectorSubcoreMesh(core_axis_name="c", subcore_axis_name="s"),
    out_type=jax.ShapeDtypeStruct((TOPK, C), jnp.float32),
    scratch_types=[
        pltpu.VMEM((_ROWS_PER_W,), jnp.int32),
        pltpu.VMEM((_ROWS_PER_W, C), jnp.float32),
        pltpu.SemaphoreType.DMA,
    ],
)
def _sc_gather(idx_hbm, x1_hbm, out_hbm, idx_v, rows_v, sem):
    wid = lax.axis_index("s") * 2 + lax.axis_index("c")

    @pl.when(wid < _SC_WORKERS)
    def _():
        base = wid * _ROWS_PER_W
        pltpu.sync_copy(idx_hbm.at[pl.ds(base, _ROWS_PER_W)], idx_v)
        pltpu.async_copy(x1_hbm.at[idx_v], rows_v, sem).wait()
        pltpu.sync_copy(rows_v, out_hbm.at[pl.ds(base, _ROWS_PER_W)])


def _global_kernel(score_ref, x1_ref, xg_ref, n2w_ref, n2b_ref, wagq_ref,
                   bagq_ref, wkv_ref, bkv_ref, wgo_ref, bgo_ref, n3w_ref,
                   n3b_ref, wf1_ref, bf1_ref, wf2_ref, bf2_ref, out_ref):
    score = score_ref[...]
    qh = jnp.dot(score.astype(jnp.bfloat16), wagq_ref[...],
                 preferred_element_type=jnp.float32) + bagq_ref[...]
    qh = qh.astype(jnp.bfloat16)
    xgn = _ln(xg_ref[...], n2w_ref[...], n2b_ref[...])
    kv = jnp.dot(xgn.astype(jnp.bfloat16), wkv_ref[...],
                 preferred_element_type=jnp.float32) + bkv_ref[...]
    kv = kv.astype(jnp.bfloat16)
    acc = jnp.zeros((TB, C), jnp.float32)
    for h in range(NH):
        q_h = qh[:, h * HD:(h + 1) * HD]
        k_h = kv[:, h * HD:(h + 1) * HD]
        v_h = kv[:, C + h * HD:C + (h + 1) * HD]
        logits = jax.lax.dot_general(
            q_h, k_h, (((1,), (1,)), ((), ())),
            preferred_element_type=jnp.float32) * SCALE
        a = jax.nn.softmax(logits, axis=-1).astype(jnp.bfloat16)
        o = jnp.dot(a, v_h, preferred_element_type=jnp.float32)
        acc = acc + jnp.dot(o.astype(jnp.bfloat16),
                            wgo_ref[h * HD:(h + 1) * HD, :],
                            preferred_element_type=jnp.float32)
    x2 = x1_ref[...] + acc + bgo_ref[...]
    xn3 = _ln(x2, n3w_ref[...], n3b_ref[...])
    f = jnp.dot(xn3.astype(jnp.bfloat16), wf1_ref[...],
                preferred_element_type=jnp.float32) + bf1_ref[...]
    f = 0.5 * f * (1.0 + jax.lax.erf(f * (1.0 / math.sqrt(2.0))))
    out_ref[...] = x2 + jnp.dot(f.astype(jnp.bfloat16), wf2_ref[...],
                                preferred_element_type=jnp.float32) + bf2_ref[...]


def kernel(x, n1_w, n1_b, qkv_w, qkv_b, al_in_w, al_in_b, al_out_w, al_out_b,
           pl_w, pl_b, n2_w, n2_b, qg_w, qg_b, kvg_w, kvg_b, ag_in_w, ag_in_b,
           ag_out_w, ag_out_b, pg_w, pg_b, n3_w, n3_b, ffn1_w, ffn1_b,
           ffn2_w, ffn2_b):
    B, Cc, H, W = x.shape
    nh_w = H // 8

    wq = al_in_w[:C] @ qkv_w[:C]
    wk = al_in_w[C:2 * C] @ qkv_w[C:2 * C]
    wv = al_in_w[2 * C:] @ qkv_w[2 * C:]
    wqkv = jnp.concatenate([wq, wk, wv], axis=0).T
    bqkv = jnp.concatenate([
        al_in_w[:C] @ qkv_b[:C] + al_in_b[:C],
        al_in_w[C:2 * C] @ qkv_b[C:2 * C] + al_in_b[C:2 * C],
        al_in_w[2 * C:] @ qkv_b[2 * C:] + al_in_b[2 * C:]])[None]
    wlo = (pl_w @ al_out_w).T
    blo = (pl_w @ al_out_b + pl_b)[None]
    wqg = qg_w.T
    bqg = qg_b[None]
    wagq = ag_in_w[:C].T
    bagq = ag_in_b[:C][None]
    wkv = jnp.concatenate([(ag_in_w[C:2 * C] @ kvg_w[:C]).T,
                           (ag_in_w[2 * C:] @ kvg_w[C:]).T], axis=1)
    bkv = jnp.concatenate([
        ag_in_w[C:2 * C] @ kvg_b[:C] + ag_in_b[C:2 * C],
        ag_in_w[2 * C:] @ kvg_b[C:] + ag_in_b[2 * C:]])[None]
    wgo = (pg_w @ ag_out_w).T
    bgo = (pg_w @ ag_out_b + pg_b)[None]
    wf1 = ffn1_w.T
    bf1 = ffn1_b[None]
    wf2 = ffn2_w.T
    bf2 = ffn2_b[None]
    bf = jnp.bfloat16
    wqkv = wqkv.astype(bf)
    wlo = wlo.astype(bf)
    wagq = wagq.astype(bf)
    wkv = wkv.astype(bf)
    wgo = wgo.astype(bf)
    wf1 = wf1.astype(bf)
    wf2 = wf2.astype(bf)
    n1w, n1b = n1_w[None], n1_b[None]
    n2w, n2b = n2_w[None], n2_b[None]
    n3w, n3b = n3_w[None], n3_b[None]

    xp = x[0].reshape(C, nh_w, 8, nh_w, 8).transpose(1, 3, 2, 4, 0)
    xp = xp.reshape(N, C)

    def full(shape):
        return pl.BlockSpec(shape, lambda i: (0,) * len(shape))

    tok = pl.BlockSpec((TB, C), lambda i: (i, 0))

    x1, score, smap = pl.pallas_call(
        _local_kernel,
        grid=(GRID,),
        in_specs=[tok, full((C, 3 * C)), full((1, 3 * C)), full((1, C)),
                  full((1, C)), full((C, C)), full((1, C)), full((1, C)),
                  full((1, C)), full((C, C)), full((1, C))],
        out_specs=[tok, pl.BlockSpec((TB, C), lambda i: (i, 0)),
                   pl.BlockSpec((1, TB // 128, 128), lambda i: (i, 0, 0))],
        out_shape=[jax.ShapeDtypeStruct((N, C), jnp.float32),
                   jax.ShapeDtypeStruct((N, C), jnp.float32),
                   jax.ShapeDtypeStruct((GRID, TB // 128, 128), jnp.float32)],
    )(xp, wqkv, bqkv, n1w, n1b, wlo, blo, n2w, n2b, wqg, bqg)
    smap = smap.reshape(N // 128, 128)

    idx = pl.pallas_call(
        _topk_kernel,
        in_specs=[pl.BlockSpec(memory_space=pltpu.VMEM)],
        out_specs=pl.BlockSpec(memory_space=pltpu.SMEM),
        out_shape=jax.ShapeDtypeStruct((1, TOPK), jnp.int32),
    )(smap)

    xg = _sc_gather(idx.reshape(TOPK), x1)

    y = pl.pallas_call(
        _global_kernel,
        grid=(GRID,),
        in_specs=[tok, pl.BlockSpec((TB, C), lambda i: (i, 0)),
                  full((TOPK, C)), full((1, C)), full((1, C)),
                  full((C, C)), full((1, C)), full((C, 2 * C)),
                  full((1, 2 * C)), full((C, C)), full((1, C)),
                  full((1, C)), full((1, C)), full((C, 2 * C)),
                  full((1, 2 * C)), full((2 * C, C)), full((1, C))],
        out_specs=pl.BlockSpec((TB, C), lambda i: (i, 0)),
        out_shape=jax.ShapeDtypeStruct((N, C), jnp.float32),
    )(score, x1, xg, n2w, n2b, wagq, bagq, wkv, bkv, wgo, bgo,
      n3w, n3b, wf1, bf1, wf2, bf2)

    out = y.reshape(nh_w, nh_w, 8, 8, C).transpose(4, 0, 2, 1, 3)
    return out.reshape(1, C, H, W)
